# Initial kernel scaffold; baseline (speedup 1.0000x reference)
import jax
import jax.numpy as jnp
from jax.experimental import pallas as pl


def kernel(inputs, adj, mask_item, item, adj_all, num_w, embedding, a_0, a_1, a_2, a_3, g_w1, g_w2, g_w3):
    raise NotImplementedError("write your pallas kernel here")


# speedup vs baseline: 2.8453x; 2.8453x over previous
import functools

import jax
import jax.numpy as jnp
from jax import lax
from jax.experimental import pallas as pl
from jax.experimental.pallas import tpu as pltpu
from jax.experimental.pallas import tpu_sc as plsc

B = 1024
L = 20
D = 128
S = 12
ALPHA = 0.2

NC = 2
NS = 16
NW = NC * NS
N_FLAT = B * L
CHUNK = N_FLAT // NW
TILE = 40
N_TILES = CHUNK // TILE
EB = 128
LANES = 16
NSEG = TILE * S // EB


def _leaky(x, slope):
    return jnp.where(x >= 0, x, slope * x)


GROUP = 8


def _make_sc_body():
    def body(inputs_hbm, item_hbm, cat_hbm, emb_hbm,
             h_out, ite_out, cat_out, ent_out,
             idx_v, iidx_v, h_v, ite_v, cat_v, ent_v,
             sem_h, sem_a, sem_e):
        wid = lax.axis_index("s") * NC + lax.axis_index("c")
        base = wid * CHUNK

        def tile_body(ti, carry):
            gbase = base + ti * TILE
            pltpu.sync_copy(inputs_hbm.at[pl.ds(gbase, TILE)], idx_v)
            pltpu.sync_copy(item_hbm.at[pl.ds(gbase, TILE)], iidx_v)
            c_h = pltpu.async_copy(emb_hbm.at[idx_v], h_v, sem_h)
            c_it = pltpu.async_copy(emb_hbm.at[iidx_v], ite_v, sem_h)
            c_cat = pltpu.async_copy(cat_hbm.at[idx_v], cat_v, sem_a)
            c_cat.wait()

            def fire(g, fcarry):
                for i in range(GROUP):
                    r = g * GROUP + i
                    pltpu.async_copy(
                        emb_hbm.at[cat_v.at[r].at[pl.ds(0, S)]],
                        ent_v.at[r], sem_e)
                return fcarry

            lax.fori_loop(0, TILE // GROUP, fire, 0)
            c_h.wait()
            c_it.wait()
            pltpu.sync_copy(h_v, h_out.at[pl.ds(gbase, TILE)])
            pltpu.sync_copy(ite_v, ite_out.at[pl.ds(gbase, TILE)])
            pltpu.sync_copy(cat_v, cat_out.at[pl.ds(gbase, TILE)])

            def drain(g, dcarry):
                pltpu.make_async_copy(
                    emb_hbm.at[pl.ds(0, 8)],
                    ent_v.at[0].at[pl.ds(0, 8)], sem_e).wait()
                return dcarry

            lax.fori_loop(0, TILE * S * D * 4 // 4096, drain, 0)
            pltpu.sync_copy(ent_v, ent_out.at[pl.ds(gbase, TILE)])
            return carry

        lax.fori_loop(0, N_TILES, tile_body, 0)

    return body


@functools.cache
def _make_sc_gather():
    return pl.kernel(
        _make_sc_body(),
        out_type=(
            jax.ShapeDtypeStruct((N_FLAT, D), jnp.float32),
            jax.ShapeDtypeStruct((N_FLAT, D), jnp.float32),
            jax.ShapeDtypeStruct((N_FLAT, 128), jnp.int32),
            jax.ShapeDtypeStruct((N_FLAT, S, D), jnp.float32),
        ),
        mesh=plsc.VectorSubcoreMesh(core_axis_name="c", subcore_axis_name="s",
                                    num_cores=NC, num_subcores=NS),
        scratch_types=[
            pltpu.VMEM((TILE,), jnp.int32),
            pltpu.VMEM((TILE,), jnp.int32),
            pltpu.VMEM((TILE, D), jnp.float32),
            pltpu.VMEM((TILE, D), jnp.float32),
            pltpu.VMEM((TILE, 128), jnp.int32),
            pltpu.VMEM((TILE, S, D), jnp.float32),
            pltpu.SemaphoreType.DMA,
            pltpu.SemaphoreType.DMA,
            pltpu.SemaphoreType.DMA,
        ],
    )


BB = 8


def _tc_body(h_ref, ite_ref, ent_ref, cat_ref, adj_ref, mask_ref,
             a_ref, w1a_ref, w1b_ref, w2_ref, w3_ref, o_ref):
    hflat = h_ref[...]
    hb = hflat.reshape(BB, L, D)
    av = a_ref[...]
    ha = hb[:, :, None, :] * av[None, None, :, :]
    e = lax.dot_general(ha.reshape(BB, L * 4, D), hb,
                        (((2,), (2,)), ((0,), (0,))))
    e = _leaky(e, ALPHA).reshape(BB, L, 4, L)
    adj = adj_ref[...]
    neg = jnp.float32(-9e15)
    att = jnp.where(adj == 1, e[:, :, 0, :], neg)
    att = jnp.where(adj == 2, e[:, :, 1, :], att)
    att = jnp.where(adj == 3, e[:, :, 2, :], att)
    att = jnp.where(adj == 4, e[:, :, 3, :], att)
    att = jax.nn.softmax(att, axis=-1)
    h_local = lax.dot_general(att, hb,
                              (((2,), (1,)), ((0,), (0,))))

    maskf = mask_ref[...].astype(jnp.float32)
    ite = ite_ref[...].reshape(BB, L, D)
    ssum = jnp.sum(ite * maskf[..., None], axis=1)
    sess = ssum / jnp.sum(maskf, axis=1)[..., None]

    catv = cat_ref[...]
    sess_pos = jnp.broadcast_to(
        sess[:, None, :], (BB, L, D)).reshape(BB * L, D)
    w1b2 = w1b_ref[...]
    ents = []
    logits = []
    for s in range(S):
        ent_s = ent_ref[:, s, :]
        wv_s = lax.bitcast_convert_type(catv[:, S + s:S + s + 1],
                                        jnp.float32)
        al_s = jnp.dot(sess_pos * ent_s, w1a_ref[...],
                       preferred_element_type=jnp.float32)
        al_s = _leaky(al_s + wv_s * w1b2, 0.2)
        logit_s = jnp.dot(al_s, w2_ref[...],
                          preferred_element_type=jnp.float32)
        ents.append(ent_s)
        logits.append(logit_s)
    m = logits[0]
    for s in range(1, S):
        m = jnp.maximum(m, logits[s])
    exps = [jnp.exp(lg - m) for lg in logits]
    denom = exps[0]
    for s in range(1, S):
        denom = denom + exps[s]
    nbr = (exps[0] / denom) * ents[0]
    for s in range(1, S):
        nbr = nbr + (exps[s] / denom) * ents[s]
    out = jnp.concatenate([hflat, nbr], axis=-1)
    hg = jnp.dot(out, w3_ref[...], preferred_element_type=jnp.float32)
    hg = jnp.maximum(hg, 0.0)
    o_ref[...] = h_local + hg.reshape(BB, L, D)


def _tc_call(h2, ite2, ent3, cat2, adj, mask, av, w1a, w1b, g_w2, g_w3,
             interpret=False):
    nblk = B // BB
    return pl.pallas_call(
        _tc_body,
        grid=(nblk,),
        in_specs=[
            pl.BlockSpec((BB * L, D), lambda b: (b, 0)),
            pl.BlockSpec((BB * L, D), lambda b: (b, 0)),
            pl.BlockSpec((BB * L, S, D), lambda b: (b, 0, 0)),
            pl.BlockSpec((BB * L, 128), lambda b: (b, 0)),
            pl.BlockSpec((BB, L, L), lambda b: (b, 0, 0)),
            pl.BlockSpec((BB, L), lambda b: (b, 0)),
            pl.BlockSpec((4, D), lambda b: (0, 0)),
            pl.BlockSpec((D, D), lambda b: (0, 0)),
            pl.BlockSpec((1, D), lambda b: (0, 0)),
            pl.BlockSpec((D, 1), lambda b: (0, 0)),
            pl.BlockSpec((2 * D, D), lambda b: (0, 0)),
        ],
        out_specs=pl.BlockSpec((BB, L, D), lambda b: (b, 0, 0)),
        out_shape=jax.ShapeDtypeStruct((B, L, D), jnp.float32),
        interpret=interpret,
    )(h2, ite2, ent3, cat2, adj, mask, av, w1a, w1b, g_w2, g_w3)


def kernel(inputs, adj, mask_item, item, adj_all, num_w, embedding,
           a_0, a_1, a_2, a_3, g_w1, g_w2, g_w3):
    flat = inputs.reshape(-1)
    itemf = item.reshape(-1)
    numw_bits = lax.bitcast_convert_type(num_w, jnp.int32)
    cat = jnp.concatenate(
        [adj_all, numw_bits,
         jnp.zeros((adj_all.shape[0], 128 - 2 * S), jnp.int32)], axis=1)
    h_flat, ite_flat, cat_rows, ent = _make_sc_gather()(
        flat, itemf, cat, embedding)

    av = jnp.concatenate([a_0, a_1, a_2, a_3], axis=1).T
    w1a = g_w1[:D]
    w1b = g_w1[D:]

    return _tc_call(h_flat, ite_flat, ent, cat_rows, adj, mask_item,
                    av, w1a, w1b, g_w2, g_w3)


# speedup vs baseline: 3.0758x; 1.0810x over previous
import functools

import jax
import jax.numpy as jnp
from jax import lax
from jax.experimental import pallas as pl
from jax.experimental.pallas import tpu as pltpu
from jax.experimental.pallas import tpu_sc as plsc

B = 1024
L = 20
D = 128
S = 12
ALPHA = 0.2

NC = 2
NS = 16
NW = NC * NS
N_FLAT = B * L
CHUNK = N_FLAT // NW
TILE = 40
N_TILES = CHUNK // TILE
EB = 128
LANES = 16
NSEG = TILE * S // EB
NCHUNK = 2


def _leaky(x, slope):
    return jnp.where(x >= 0, x, slope * x)


GROUP = 8


def _make_sc_body(nflat_c, coff):
    per_worker = nflat_c // NW
    n_tiles = per_worker // TILE

    def body(inputs_hbm, item_hbm, cat_hbm, emb_hbm,
             h_out, ite_out, cat_out, ent_out,
             idx_v, iidx_v, h_v, ite_v, cat_v, ent_v,
             sem_h, sem_a, sem_e):
        wid = lax.axis_index("s") * NC + lax.axis_index("c")
        base = wid * per_worker

        def tile_body(ti, carry):
            gbase = base + ti * TILE
            pltpu.sync_copy(inputs_hbm.at[pl.ds(coff + gbase, TILE)], idx_v)
            pltpu.sync_copy(item_hbm.at[pl.ds(coff + gbase, TILE)], iidx_v)
            c_h = pltpu.async_copy(emb_hbm.at[idx_v], h_v, sem_h)
            c_it = pltpu.async_copy(emb_hbm.at[iidx_v], ite_v, sem_h)
            c_cat = pltpu.async_copy(cat_hbm.at[idx_v], cat_v, sem_a)
            c_cat.wait()

            def fire(g, fcarry):
                for i in range(GROUP):
                    r = g * GROUP + i
                    pltpu.async_copy(
                        emb_hbm.at[cat_v.at[r].at[pl.ds(0, S)]],
                        ent_v.at[r], sem_e)
                return fcarry

            lax.fori_loop(0, TILE // GROUP, fire, 0)
            c_h.wait()
            c_it.wait()
            pltpu.sync_copy(h_v, h_out.at[pl.ds(gbase, TILE)])
            pltpu.sync_copy(ite_v, ite_out.at[pl.ds(gbase, TILE)])
            pltpu.sync_copy(cat_v, cat_out.at[pl.ds(gbase, TILE)])

            def drain(g, dcarry):
                pltpu.make_async_copy(
                    emb_hbm.at[pl.ds(0, 8)],
                    ent_v.at[0].at[pl.ds(0, 8)], sem_e).wait()
                return dcarry

            lax.fori_loop(0, TILE * S * D * 4 // 4096, drain, 0)
            pltpu.sync_copy(ent_v, ent_out.at[pl.ds(gbase, TILE)])
            return carry

        lax.fori_loop(0, n_tiles, tile_body, 0)

    return body


@functools.cache
def _make_sc_gather(nflat_c, coff):
    return pl.kernel(
        _make_sc_body(nflat_c, coff),
        out_type=(
            jax.ShapeDtypeStruct((nflat_c, D), jnp.float32),
            jax.ShapeDtypeStruct((nflat_c, D), jnp.float32),
            jax.ShapeDtypeStruct((nflat_c, 128), jnp.int32),
            jax.ShapeDtypeStruct((nflat_c, S, D), jnp.float32),
        ),
        mesh=plsc.VectorSubcoreMesh(core_axis_name="c", subcore_axis_name="s",
                                    num_cores=NC, num_subcores=NS),
        scratch_types=[
            pltpu.VMEM((TILE,), jnp.int32),
            pltpu.VMEM((TILE,), jnp.int32),
            pltpu.VMEM((TILE, D), jnp.float32),
            pltpu.VMEM((TILE, D), jnp.float32),
            pltpu.VMEM((TILE, 128), jnp.int32),
            pltpu.VMEM((TILE, S, D), jnp.float32),
            pltpu.SemaphoreType.DMA,
            pltpu.SemaphoreType.DMA,
            pltpu.SemaphoreType.DMA,
        ],
    )


BB = 8


def _tc_body(h_ref, ite_ref, ent_ref, cat_ref, adj_ref, mask_ref,
             a_ref, w1a_ref, w1b_ref, w2_ref, w3_ref, o_ref):
    hflat = h_ref[...]
    hb = hflat.reshape(BB, L, D)
    av = a_ref[...]
    ha = hb[:, :, None, :] * av[None, None, :, :]
    e = lax.dot_general(ha.reshape(BB, L * 4, D), hb,
                        (((2,), (2,)), ((0,), (0,))))
    e = _leaky(e, ALPHA).reshape(BB, L, 4, L)
    adj = adj_ref[...]
    neg = jnp.float32(-9e15)
    att = jnp.where(adj == 1, e[:, :, 0, :], neg)
    att = jnp.where(adj == 2, e[:, :, 1, :], att)
    att = jnp.where(adj == 3, e[:, :, 2, :], att)
    att = jnp.where(adj == 4, e[:, :, 3, :], att)
    att = jax.nn.softmax(att, axis=-1)
    h_local = lax.dot_general(att, hb,
                              (((2,), (1,)), ((0,), (0,))))

    maskf = mask_ref[...].astype(jnp.float32)
    ite = ite_ref[...].reshape(BB, L, D)
    ssum = jnp.sum(ite * maskf[..., None], axis=1)
    sess = ssum / jnp.sum(maskf, axis=1)[..., None]

    catv = cat_ref[...]
    sess_pos = jnp.broadcast_to(
        sess[:, None, :], (BB, L, D)).reshape(BB * L, D)
    w1b2 = w1b_ref[...]
    ents = []
    logits = []
    for s in range(S):
        ent_s = ent_ref[:, s, :]
        wv_s = lax.bitcast_convert_type(catv[:, S + s:S + s + 1],
                                        jnp.float32)
        al_s = jnp.dot(sess_pos * ent_s, w1a_ref[...],
                       preferred_element_type=jnp.float32)
        al_s = _leaky(al_s + wv_s * w1b2, 0.2)
        logit_s = jnp.dot(al_s, w2_ref[...],
                          preferred_element_type=jnp.float32)
        ents.append(ent_s)
        logits.append(logit_s)
    m = logits[0]
    for s in range(1, S):
        m = jnp.maximum(m, logits[s])
    exps = [jnp.exp(lg - m) for lg in logits]
    denom = exps[0]
    for s in range(1, S):
        denom = denom + exps[s]
    nbr = (exps[0] / denom) * ents[0]
    for s in range(1, S):
        nbr = nbr + (exps[s] / denom) * ents[s]
    out = jnp.concatenate([hflat, nbr], axis=-1)
    hg = jnp.dot(out, w3_ref[...], preferred_element_type=jnp.float32)
    hg = jnp.maximum(hg, 0.0)
    o_ref[...] = h_local + hg.reshape(BB, L, D)


def _tc_call(h2, ite2, ent3, cat2, adj, mask, av, w1a, w1b, g_w2, g_w3,
             bc=B, boff=0, interpret=False):
    nblk = bc // BB
    ob = boff // BB
    return pl.pallas_call(
        _tc_body,
        grid=(nblk,),
        in_specs=[
            pl.BlockSpec((BB * L, D), lambda b: (b, 0)),
            pl.BlockSpec((BB * L, D), lambda b: (b, 0)),
            pl.BlockSpec((BB * L, S, D), lambda b: (b, 0, 0)),
            pl.BlockSpec((BB * L, 128), lambda b: (b, 0)),
            pl.BlockSpec((BB, L, L), lambda b, _o=ob: (b + _o, 0, 0)),
            pl.BlockSpec((BB, L), lambda b, _o=ob: (b + _o, 0)),
            pl.BlockSpec((4, D), lambda b: (0, 0)),
            pl.BlockSpec((D, D), lambda b: (0, 0)),
            pl.BlockSpec((1, D), lambda b: (0, 0)),
            pl.BlockSpec((D, 1), lambda b: (0, 0)),
            pl.BlockSpec((2 * D, D), lambda b: (0, 0)),
        ],
        out_specs=pl.BlockSpec((BB, L, D), lambda b: (b, 0, 0)),
        out_shape=jax.ShapeDtypeStruct((bc, L, D), jnp.float32),
        interpret=interpret,
    )(h2, ite2, ent3, cat2, adj, mask, av, w1a, w1b, g_w2, g_w3)


def kernel(inputs, adj, mask_item, item, adj_all, num_w, embedding,
           a_0, a_1, a_2, a_3, g_w1, g_w2, g_w3):
    flat = inputs.reshape(-1)
    itemf = item.reshape(-1)
    numw_bits = lax.bitcast_convert_type(num_w, jnp.int32)
    cat = jnp.concatenate(
        [adj_all, numw_bits,
         jnp.zeros((adj_all.shape[0], 128 - 2 * S), jnp.int32)], axis=1)
    av = jnp.concatenate([a_0, a_1, a_2, a_3], axis=1).T
    w1a = g_w1[:D]
    w1b = g_w1[D:]

    bc = B // NCHUNK
    nflat_c = bc * L
    outs = []
    for c in range(NCHUNK):
        h_c, ite_c, cat_c, ent_c = _make_sc_gather(nflat_c, c * nflat_c)(
            flat, itemf, cat, embedding)
        outs.append(_tc_call(h_c, ite_c, ent_c, cat_c, adj, mask_item,
                             av, w1a, w1b, g_w2, g_w3,
                             bc=bc, boff=c * bc))
    if NCHUNK == 1:
        return outs[0]
    return jnp.concatenate(outs, axis=0)


# speedup vs baseline: 3.0960x; 1.0066x over previous
import functools

import jax
import jax.numpy as jnp
from jax import lax
from jax.experimental import pallas as pl
from jax.experimental.pallas import tpu as pltpu
from jax.experimental.pallas import tpu_sc as plsc

B = 1024
L = 20
D = 128
S = 12
ALPHA = 0.2

NC = 2
NS = 16
NW = NC * NS
N_FLAT = B * L
CHUNK = N_FLAT // NW
TILE = 40
N_TILES = CHUNK // TILE
EB = 128
LANES = 16
NSEG = TILE * S // EB
NCHUNK = 2


def _leaky(x, slope):
    return jnp.maximum(x, slope * x)


GROUP = 8


def _make_sc_body(nflat_c, coff):
    per_worker = nflat_c // NW
    n_tiles = per_worker // TILE

    def body(inputs_hbm, item_hbm, cat_hbm, emb_hbm,
             h_out, ite_out, cat_out, ent_out,
             idx_v, iidx_v, h_v, ite_v, cat_v, ent_v,
             sem_h, sem_a, sem_e):
        wid = lax.axis_index("s") * NC + lax.axis_index("c")
        base = wid * per_worker

        def tile_body(ti, carry):
            gbase = base + ti * TILE
            pltpu.sync_copy(inputs_hbm.at[pl.ds(coff + gbase, TILE)], idx_v)
            pltpu.sync_copy(item_hbm.at[pl.ds(coff + gbase, TILE)], iidx_v)
            c_h = pltpu.async_copy(emb_hbm.at[idx_v], h_v, sem_h)
            c_it = pltpu.async_copy(emb_hbm.at[iidx_v], ite_v, sem_h)
            c_cat = pltpu.async_copy(cat_hbm.at[idx_v], cat_v, sem_a)
            c_cat.wait()

            def fire(g, fcarry):
                for i in range(GROUP):
                    r = g * GROUP + i
                    pltpu.async_copy(
                        emb_hbm.at[cat_v.at[r].at[pl.ds(0, S)]],
                        ent_v.at[r], sem_e)
                return fcarry

            lax.fori_loop(0, TILE // GROUP, fire, 0)
            c_h.wait()
            c_it.wait()
            pltpu.sync_copy(h_v, h_out.at[pl.ds(gbase, TILE)])
            pltpu.sync_copy(ite_v, ite_out.at[pl.ds(gbase, TILE)])
            pltpu.sync_copy(cat_v, cat_out.at[pl.ds(gbase, TILE)])

            def drain(g, dcarry):
                pltpu.make_async_copy(
                    emb_hbm.at[pl.ds(0, 8)],
                    ent_v.at[0].at[pl.ds(0, 8)], sem_e).wait()
                return dcarry

            lax.fori_loop(0, TILE * S * D * 4 // 4096, drain, 0)
            pltpu.sync_copy(ent_v, ent_out.at[pl.ds(gbase, TILE)])
            return carry

        lax.fori_loop(0, n_tiles, tile_body, 0)

    return body


@functools.cache
def _make_sc_gather(nflat_c, coff):
    return pl.kernel(
        _make_sc_body(nflat_c, coff),
        out_type=(
            jax.ShapeDtypeStruct((nflat_c, D), jnp.float32),
            jax.ShapeDtypeStruct((nflat_c, D), jnp.float32),
            jax.ShapeDtypeStruct((nflat_c, 128), jnp.int32),
            jax.ShapeDtypeStruct((nflat_c, S, D), jnp.float32),
        ),
        mesh=plsc.VectorSubcoreMesh(core_axis_name="c", subcore_axis_name="s",
                                    num_cores=NC, num_subcores=NS),
        scratch_types=[
            pltpu.VMEM((TILE,), jnp.int32),
            pltpu.VMEM((TILE,), jnp.int32),
            pltpu.VMEM((TILE, D), jnp.float32),
            pltpu.VMEM((TILE, D), jnp.float32),
            pltpu.VMEM((TILE, 128), jnp.int32),
            pltpu.VMEM((TILE, S, D), jnp.float32),
            pltpu.SemaphoreType.DMA,
            pltpu.SemaphoreType.DMA,
            pltpu.SemaphoreType.DMA,
        ],
    )


BB = 8


def _tc_body(h_ref, ite_ref, ent_ref, cat_ref, adj_ref, mask_ref,
             a_ref, w1a_ref, w1b_ref, w2_ref, w3_ref, o_ref):
    hflat = h_ref[...]
    hb = hflat.reshape(BB, L, D)
    av = a_ref[...]
    ha = hb[:, :, None, :] * av[None, None, :, :]
    e = lax.dot_general(ha.reshape(BB, L * 4, D), hb,
                        (((2,), (2,)), ((0,), (0,))))
    e = _leaky(e, ALPHA).reshape(BB, L, 4, L)
    adj = adj_ref[...]
    neg = jnp.float32(-9e15)
    att = jnp.where(adj == 1, e[:, :, 0, :], neg)
    att = jnp.where(adj == 2, e[:, :, 1, :], att)
    att = jnp.where(adj == 3, e[:, :, 2, :], att)
    att = jnp.where(adj == 4, e[:, :, 3, :], att)
    att = jax.nn.softmax(att, axis=-1)
    h_local = lax.dot_general(att, hb,
                              (((2,), (1,)), ((0,), (0,))))

    maskf = mask_ref[...].astype(jnp.float32)
    ite = ite_ref[...].reshape(BB, L, D)
    ssum = jnp.sum(ite * maskf[..., None], axis=1)
    sess = ssum / jnp.sum(maskf, axis=1)[..., None]

    catv = cat_ref[...]
    sess_pos = jnp.broadcast_to(
        sess[:, None, :], (BB, L, D)).reshape(BB * L, D)
    w1b2 = w1b_ref[...]
    ents = []
    logits = []
    for s in range(S):
        ent_s = ent_ref[:, s, :]
        wv_s = lax.bitcast_convert_type(catv[:, S + s:S + s + 1],
                                        jnp.float32)
        al_s = jnp.dot(sess_pos * ent_s, w1a_ref[...],
                       preferred_element_type=jnp.float32)
        al_s = _leaky(al_s + wv_s * w1b2, 0.2)
        logit_s = jnp.dot(al_s, w2_ref[...],
                          preferred_element_type=jnp.float32)
        ents.append(ent_s)
        logits.append(logit_s)
    m = logits[0]
    for s in range(1, S):
        m = jnp.maximum(m, logits[s])
    exps = [jnp.exp(lg - m) for lg in logits]
    denom = exps[0]
    for s in range(1, S):
        denom = denom + exps[s]
    nbr = (exps[0] / denom) * ents[0]
    for s in range(1, S):
        nbr = nbr + (exps[s] / denom) * ents[s]
    out = jnp.concatenate([hflat, nbr], axis=-1)
    hg = jnp.dot(out, w3_ref[...], preferred_element_type=jnp.float32)
    hg = jnp.maximum(hg, 0.0)
    o_ref[...] = h_local + hg.reshape(BB, L, D)


def _tc_call(h2, ite2, ent3, cat2, adj, mask, av, w1a, w1b, g_w2, g_w3,
             bc=B, boff=0, interpret=False):
    nblk = bc // BB
    ob = boff // BB
    return pl.pallas_call(
        _tc_body,
        grid=(nblk,),
        in_specs=[
            pl.BlockSpec((BB * L, D), lambda b: (b, 0)),
            pl.BlockSpec((BB * L, D), lambda b: (b, 0)),
            pl.BlockSpec((BB * L, S, D), lambda b: (b, 0, 0)),
            pl.BlockSpec((BB * L, 128), lambda b: (b, 0)),
            pl.BlockSpec((BB, L, L), lambda b, _o=ob: (b + _o, 0, 0)),
            pl.BlockSpec((BB, L), lambda b, _o=ob: (b + _o, 0)),
            pl.BlockSpec((4, D), lambda b: (0, 0)),
            pl.BlockSpec((D, D), lambda b: (0, 0)),
            pl.BlockSpec((1, D), lambda b: (0, 0)),
            pl.BlockSpec((D, 1), lambda b: (0, 0)),
            pl.BlockSpec((2 * D, D), lambda b: (0, 0)),
        ],
        out_specs=pl.BlockSpec((BB, L, D), lambda b: (b, 0, 0)),
        out_shape=jax.ShapeDtypeStruct((bc, L, D), jnp.float32),
        interpret=interpret,
    )(h2, ite2, ent3, cat2, adj, mask, av, w1a, w1b, g_w2, g_w3)


def kernel(inputs, adj, mask_item, item, adj_all, num_w, embedding,
           a_0, a_1, a_2, a_3, g_w1, g_w2, g_w3):
    flat = inputs.reshape(-1)
    itemf = item.reshape(-1)
    numw_bits = lax.bitcast_convert_type(num_w, jnp.int32)
    cat = jnp.concatenate(
        [adj_all, numw_bits,
         jnp.zeros((adj_all.shape[0], 128 - 2 * S), jnp.int32)], axis=1)
    av = jnp.concatenate([a_0, a_1, a_2, a_3], axis=1).T
    w1a = g_w1[:D]
    w1b = g_w1[D:]

    bc = B // NCHUNK
    nflat_c = bc * L
    outs = []
    for c in range(NCHUNK):
        h_c, ite_c, cat_c, ent_c = _make_sc_gather(nflat_c, c * nflat_c)(
            flat, itemf, cat, embedding)
        outs.append(_tc_call(h_c, ite_c, ent_c, cat_c, adj, mask_item,
                             av, w1a, w1b, g_w2, g_w3,
                             bc=bc, boff=c * bc))
    if NCHUNK == 1:
        return outs[0]
    return jnp.concatenate(outs, axis=0)


# speedup vs baseline: 3.2664x; 1.0550x over previous
import functools

import jax
import jax.numpy as jnp
from jax import lax
from jax.experimental import pallas as pl
from jax.experimental.pallas import tpu as pltpu
from jax.experimental.pallas import tpu_sc as plsc

B = 1024
L = 20
D = 128
S = 12
ALPHA = 0.2

NC = 2
NS = 16
NW = NC * NS
N_FLAT = B * L
CHUNK = N_FLAT // NW
TILE = 40
N_TILES = CHUNK // TILE
EB = 128
LANES = 16
NSEG = TILE * S // EB
NCHUNK = 4


def _leaky(x, slope):
    return jnp.maximum(x, slope * x)


GROUP = 8


def _make_sc_body(nflat_c, coff):
    per_worker = nflat_c // NW
    n_tiles = per_worker // TILE

    def body(inputs_hbm, item_hbm, cat_hbm, emb_hbm,
             h_out, ite_out, cat_out, ent_out,
             idx_v, iidx_v, h_v, ite_v, cat_v, ent_v,
             sem_h, sem_a, sem_e):
        wid = lax.axis_index("s") * NC + lax.axis_index("c")
        base = wid * per_worker

        def tile_body(ti, carry):
            gbase = base + ti * TILE
            pltpu.sync_copy(inputs_hbm.at[pl.ds(coff + gbase, TILE)], idx_v)
            pltpu.sync_copy(item_hbm.at[pl.ds(coff + gbase, TILE)], iidx_v)
            c_h = pltpu.async_copy(emb_hbm.at[idx_v], h_v, sem_h)
            c_it = pltpu.async_copy(emb_hbm.at[iidx_v], ite_v, sem_h)
            c_cat = pltpu.async_copy(cat_hbm.at[idx_v], cat_v, sem_a)
            c_cat.wait()

            def fire(g, fcarry):
                for i in range(GROUP):
                    r = g * GROUP + i
                    pltpu.async_copy(
                        emb_hbm.at[cat_v.at[r].at[pl.ds(0, S)]],
                        ent_v.at[r], sem_e)
                return fcarry

            lax.fori_loop(0, TILE // GROUP, fire, 0)
            c_h.wait()
            c_it.wait()
            pltpu.sync_copy(h_v, h_out.at[pl.ds(gbase, TILE)])
            pltpu.sync_copy(ite_v, ite_out.at[pl.ds(gbase, TILE)])
            pltpu.sync_copy(cat_v, cat_out.at[pl.ds(gbase, TILE)])

            def drain(g, dcarry):
                pltpu.make_async_copy(
                    emb_hbm.at[pl.ds(0, 8)],
                    ent_v.at[0].at[pl.ds(0, 8)], sem_e).wait()
                return dcarry

            lax.fori_loop(0, TILE * S * D * 4 // 4096, drain, 0)
            pltpu.sync_copy(ent_v, ent_out.at[pl.ds(gbase, TILE)])
            return carry

        lax.fori_loop(0, n_tiles, tile_body, 0)

    return body


@functools.cache
def _make_sc_gather(nflat_c, coff):
    return pl.kernel(
        _make_sc_body(nflat_c, coff),
        out_type=(
            jax.ShapeDtypeStruct((nflat_c, D), jnp.float32),
            jax.ShapeDtypeStruct((nflat_c, D), jnp.float32),
            jax.ShapeDtypeStruct((nflat_c, 128), jnp.int32),
            jax.ShapeDtypeStruct((nflat_c, S, D), jnp.float32),
        ),
        mesh=plsc.VectorSubcoreMesh(core_axis_name="c", subcore_axis_name="s",
                                    num_cores=NC, num_subcores=NS),
        scratch_types=[
            pltpu.VMEM((TILE,), jnp.int32),
            pltpu.VMEM((TILE,), jnp.int32),
            pltpu.VMEM((TILE, D), jnp.float32),
            pltpu.VMEM((TILE, D), jnp.float32),
            pltpu.VMEM((TILE, 128), jnp.int32),
            pltpu.VMEM((TILE, S, D), jnp.float32),
            pltpu.SemaphoreType.DMA,
            pltpu.SemaphoreType.DMA,
            pltpu.SemaphoreType.DMA,
        ],
    )


BB = 8


def _tc_body(h_ref, ite_ref, ent_ref, cat_ref, adj_ref, mask_ref,
             a_ref, w1a_ref, w1b_ref, w2_ref, w3_ref, o_ref):
    hflat = h_ref[...]
    hb = hflat.reshape(BB, L, D)
    av = a_ref[...]
    ha = hb[:, :, None, :] * av[None, None, :, :]
    e = lax.dot_general(ha.reshape(BB, L * 4, D), hb,
                        (((2,), (2,)), ((0,), (0,))))
    e = _leaky(e, ALPHA).reshape(BB, L, 4, L)
    adj = adj_ref[...]
    neg = jnp.float32(-9e15)
    att = jnp.where(adj == 1, e[:, :, 0, :], neg)
    att = jnp.where(adj == 2, e[:, :, 1, :], att)
    att = jnp.where(adj == 3, e[:, :, 2, :], att)
    att = jnp.where(adj == 4, e[:, :, 3, :], att)
    att = jax.nn.softmax(att, axis=-1)
    h_local = lax.dot_general(att, hb,
                              (((2,), (1,)), ((0,), (0,))))

    maskf = mask_ref[...].astype(jnp.float32)
    ite = ite_ref[...].reshape(BB, L, D)
    ssum = jnp.sum(ite * maskf[..., None], axis=1)
    sess = ssum / jnp.sum(maskf, axis=1)[..., None]

    catv = cat_ref[...]
    sess_pos = jnp.broadcast_to(
        sess[:, None, :], (BB, L, D)).reshape(BB * L, D)
    w1b2 = w1b_ref[...]
    ents = []
    logits = []
    for s in range(S):
        ent_s = ent_ref[:, s, :]
        wv_s = lax.bitcast_convert_type(catv[:, S + s:S + s + 1],
                                        jnp.float32)
        al_s = jnp.dot(sess_pos * ent_s, w1a_ref[...],
                       preferred_element_type=jnp.float32)
        al_s = _leaky(al_s + wv_s * w1b2, 0.2)
        logit_s = jnp.dot(al_s, w2_ref[...],
                          preferred_element_type=jnp.float32)
        ents.append(ent_s)
        logits.append(logit_s)
    m = logits[0]
    for s in range(1, S):
        m = jnp.maximum(m, logits[s])
    exps = [jnp.exp(lg - m) for lg in logits]
    denom = exps[0]
    for s in range(1, S):
        denom = denom + exps[s]
    nbr = (exps[0] / denom) * ents[0]
    for s in range(1, S):
        nbr = nbr + (exps[s] / denom) * ents[s]
    out = jnp.concatenate([hflat, nbr], axis=-1)
    hg = jnp.dot(out, w3_ref[...], preferred_element_type=jnp.float32)
    hg = jnp.maximum(hg, 0.0)
    o_ref[...] = h_local + hg.reshape(BB, L, D)


def _tc_call(h2, ite2, ent3, cat2, adj, mask, av, w1a, w1b, g_w2, g_w3,
             bc=B, boff=0, interpret=False):
    nblk = bc // BB
    ob = boff // BB
    return pl.pallas_call(
        _tc_body,
        grid=(nblk,),
        in_specs=[
            pl.BlockSpec((BB * L, D), lambda b: (b, 0)),
            pl.BlockSpec((BB * L, D), lambda b: (b, 0)),
            pl.BlockSpec((BB * L, S, D), lambda b: (b, 0, 0)),
            pl.BlockSpec((BB * L, 128), lambda b: (b, 0)),
            pl.BlockSpec((BB, L, L), lambda b, _o=ob: (b + _o, 0, 0)),
            pl.BlockSpec((BB, L), lambda b, _o=ob: (b + _o, 0)),
            pl.BlockSpec((4, D), lambda b: (0, 0)),
            pl.BlockSpec((D, D), lambda b: (0, 0)),
            pl.BlockSpec((1, D), lambda b: (0, 0)),
            pl.BlockSpec((D, 1), lambda b: (0, 0)),
            pl.BlockSpec((2 * D, D), lambda b: (0, 0)),
        ],
        out_specs=pl.BlockSpec((BB, L, D), lambda b: (b, 0, 0)),
        out_shape=jax.ShapeDtypeStruct((bc, L, D), jnp.float32),
        interpret=interpret,
    )(h2, ite2, ent3, cat2, adj, mask, av, w1a, w1b, g_w2, g_w3)


def kernel(inputs, adj, mask_item, item, adj_all, num_w, embedding,
           a_0, a_1, a_2, a_3, g_w1, g_w2, g_w3):
    flat = inputs.reshape(-1)
    itemf = item.reshape(-1)
    numw_bits = lax.bitcast_convert_type(num_w, jnp.int32)
    cat = jnp.concatenate(
        [adj_all, numw_bits,
         jnp.zeros((adj_all.shape[0], 128 - 2 * S), jnp.int32)], axis=1)
    av = jnp.concatenate([a_0, a_1, a_2, a_3], axis=1).T
    w1a = g_w1[:D]
    w1b = g_w1[D:]

    bc = B // NCHUNK
    nflat_c = bc * L
    outs = []
    for c in range(NCHUNK):
        h_c, ite_c, cat_c, ent_c = _make_sc_gather(nflat_c, c * nflat_c)(
            flat, itemf, cat, embedding)
        outs.append(_tc_call(h_c, ite_c, ent_c, cat_c, adj, mask_item,
                             av, w1a, w1b, g_w2, g_w3,
                             bc=bc, boff=c * bc))
    if NCHUNK == 1:
        return outs[0]
    return jnp.concatenate(outs, axis=0)


# speedup vs baseline: 3.4622x; 1.0599x over previous
import functools

import jax
import jax.numpy as jnp
from jax import lax
from jax.experimental import pallas as pl
from jax.experimental.pallas import tpu as pltpu
from jax.experimental.pallas import tpu_sc as plsc

B = 1024
L = 20
D = 128
S = 12
ALPHA = 0.2

NC = 2
NS = 16
NW = NC * NS
N_FLAT = B * L
CHUNK = N_FLAT // NW
TILE = 40
N_TILES = CHUNK // TILE
EB = 128
LANES = 16
NSEG = TILE * S // EB
NCHUNK = 4


def _leaky(x, slope):
    return jnp.maximum(x, slope * x)


GROUP = 8


def _make_sc_body(nflat_c, coff):
    per_worker = nflat_c // NW
    n_tiles = per_worker // TILE

    def body(inputs_hbm, item_hbm, cat_hbm, emb_hbm,
             h_out, ite_out, cat_out, ent_out,
             idx_v, iidx_v, h_v, ite_v, cat_v, ent_v,
             sem_h, sem_a, sem_e):
        wid = lax.axis_index("s") * NC + lax.axis_index("c")
        base = wid * per_worker

        def tile_body(ti, carry):
            gbase = base + ti * TILE
            pltpu.sync_copy(inputs_hbm.at[pl.ds(coff + gbase, TILE)], idx_v)
            pltpu.sync_copy(item_hbm.at[pl.ds(coff + gbase, TILE)], iidx_v)
            c_h = pltpu.async_copy(emb_hbm.at[idx_v], h_v, sem_h)
            c_it = pltpu.async_copy(emb_hbm.at[iidx_v], ite_v, sem_h)
            c_cat = pltpu.async_copy(cat_hbm.at[idx_v], cat_v, sem_a)
            c_cat.wait()

            def fire(g, fcarry):
                for i in range(GROUP):
                    r = g * GROUP + i
                    pltpu.async_copy(
                        emb_hbm.at[cat_v.at[r].at[pl.ds(0, S)]],
                        ent_v.at[r], sem_e)
                return fcarry

            lax.fori_loop(0, TILE // GROUP, fire, 0)
            c_h.wait()
            c_it.wait()
            pltpu.sync_copy(h_v, h_out.at[pl.ds(gbase, TILE)])
            pltpu.sync_copy(ite_v, ite_out.at[pl.ds(gbase, TILE)])
            pltpu.sync_copy(cat_v, cat_out.at[pl.ds(gbase, TILE)])

            def drain(g, dcarry):
                pltpu.make_async_copy(
                    emb_hbm.at[pl.ds(0, 8)],
                    ent_v.at[0].at[pl.ds(0, 8)], sem_e).wait()
                return dcarry

            lax.fori_loop(0, TILE * S * D * 4 // 4096, drain, 0)
            pltpu.sync_copy(ent_v, ent_out.at[pl.ds(gbase, TILE)])
            return carry

        lax.fori_loop(0, n_tiles, tile_body, 0)

    return body


@functools.cache
def _make_sc_gather(nflat_c, coff):
    return pl.kernel(
        _make_sc_body(nflat_c, coff),
        out_type=(
            jax.ShapeDtypeStruct((nflat_c, D), jnp.float32),
            jax.ShapeDtypeStruct((nflat_c, D), jnp.float32),
            jax.ShapeDtypeStruct((nflat_c, 128), jnp.int32),
            jax.ShapeDtypeStruct((nflat_c, S, D), jnp.float32),
        ),
        mesh=plsc.VectorSubcoreMesh(core_axis_name="c", subcore_axis_name="s",
                                    num_cores=NC, num_subcores=NS),
        scratch_types=[
            pltpu.VMEM((TILE,), jnp.int32),
            pltpu.VMEM((TILE,), jnp.int32),
            pltpu.VMEM((TILE, D), jnp.float32),
            pltpu.VMEM((TILE, D), jnp.float32),
            pltpu.VMEM((TILE, 128), jnp.int32),
            pltpu.VMEM((TILE, S, D), jnp.float32),
            pltpu.SemaphoreType.DMA,
            pltpu.SemaphoreType.DMA,
            pltpu.SemaphoreType.DMA,
        ],
    )


BB = 16


def _tc_body(h_ref, ite_ref, ent_ref, cat_ref, adj_ref, mask_ref,
             a_ref, w1a_ref, w1b_ref, w2_ref, w3_ref, o_ref):
    hflat = h_ref[...]
    hb = hflat.reshape(BB, L, D)
    av = a_ref[...]
    ha = hb[:, :, None, :] * av[None, None, :, :]
    e = lax.dot_general(ha.reshape(BB, L * 4, D), hb,
                        (((2,), (2,)), ((0,), (0,))))
    e = _leaky(e, ALPHA).reshape(BB, L, 4, L)
    adj = adj_ref[...]
    neg = jnp.float32(-9e15)
    att = jnp.where(adj == 1, e[:, :, 0, :], neg)
    att = jnp.where(adj == 2, e[:, :, 1, :], att)
    att = jnp.where(adj == 3, e[:, :, 2, :], att)
    att = jnp.where(adj == 4, e[:, :, 3, :], att)
    att = jax.nn.softmax(att, axis=-1)
    h_local = lax.dot_general(att, hb,
                              (((2,), (1,)), ((0,), (0,))))

    maskf = mask_ref[...].astype(jnp.float32)
    ite = ite_ref[...].reshape(BB, L, D)
    ssum = jnp.sum(ite * maskf[..., None], axis=1)
    sess = ssum / jnp.sum(maskf, axis=1)[..., None]

    catv = cat_ref[...]
    sess_pos = jnp.broadcast_to(
        sess[:, None, :], (BB, L, D)).reshape(BB * L, D)
    w1b2 = w1b_ref[...]
    ents = []
    logits = []
    for s in range(S):
        ent_s = ent_ref[:, s, :]
        wv_s = lax.bitcast_convert_type(catv[:, S + s:S + s + 1],
                                        jnp.float32)
        al_s = jnp.dot(sess_pos * ent_s, w1a_ref[...],
                       preferred_element_type=jnp.float32)
        al_s = _leaky(al_s + wv_s * w1b2, 0.2)
        logit_s = jnp.dot(al_s, w2_ref[...],
                          preferred_element_type=jnp.float32)
        ents.append(ent_s)
        logits.append(logit_s)
    m = logits[0]
    for s in range(1, S):
        m = jnp.maximum(m, logits[s])
    exps = [jnp.exp(lg - m) for lg in logits]
    denom = exps[0]
    for s in range(1, S):
        denom = denom + exps[s]
    nbr = (exps[0] / denom) * ents[0]
    for s in range(1, S):
        nbr = nbr + (exps[s] / denom) * ents[s]
    out = jnp.concatenate([hflat, nbr], axis=-1)
    hg = jnp.dot(out, w3_ref[...], preferred_element_type=jnp.float32)
    hg = jnp.maximum(hg, 0.0)
    o_ref[...] = h_local + hg.reshape(BB, L, D)


def _tc_call(h2, ite2, ent3, cat2, adj, mask, av, w1a, w1b, g_w2, g_w3,
             bc=B, boff=0, interpret=False):
    nblk = bc // BB
    ob = boff // BB
    return pl.pallas_call(
        _tc_body,
        grid=(nblk,),
        in_specs=[
            pl.BlockSpec((BB * L, D), lambda b: (b, 0)),
            pl.BlockSpec((BB * L, D), lambda b: (b, 0)),
            pl.BlockSpec((BB * L, S, D), lambda b: (b, 0, 0)),
            pl.BlockSpec((BB * L, 128), lambda b: (b, 0)),
            pl.BlockSpec((BB, L, L), lambda b, _o=ob: (b + _o, 0, 0)),
            pl.BlockSpec((BB, L), lambda b, _o=ob: (b + _o, 0)),
            pl.BlockSpec((4, D), lambda b: (0, 0)),
            pl.BlockSpec((D, D), lambda b: (0, 0)),
            pl.BlockSpec((1, D), lambda b: (0, 0)),
            pl.BlockSpec((D, 1), lambda b: (0, 0)),
            pl.BlockSpec((2 * D, D), lambda b: (0, 0)),
        ],
        out_specs=pl.BlockSpec((BB, L, D), lambda b: (b, 0, 0)),
        out_shape=jax.ShapeDtypeStruct((bc, L, D), jnp.float32),
        interpret=interpret,
    )(h2, ite2, ent3, cat2, adj, mask, av, w1a, w1b, g_w2, g_w3)


def kernel(inputs, adj, mask_item, item, adj_all, num_w, embedding,
           a_0, a_1, a_2, a_3, g_w1, g_w2, g_w3):
    flat = inputs.reshape(-1)
    itemf = item.reshape(-1)
    numw_bits = lax.bitcast_convert_type(num_w, jnp.int32)
    cat = jnp.concatenate(
        [adj_all, numw_bits,
         jnp.zeros((adj_all.shape[0], 128 - 2 * S), jnp.int32)], axis=1)
    av = jnp.concatenate([a_0, a_1, a_2, a_3], axis=1).T
    w1a = g_w1[:D]
    w1b = g_w1[D:]

    bc = B // NCHUNK
    nflat_c = bc * L
    outs = []
    for c in range(NCHUNK):
        h_c, ite_c, cat_c, ent_c = _make_sc_gather(nflat_c, c * nflat_c)(
            flat, itemf, cat, embedding)
        outs.append(_tc_call(h_c, ite_c, ent_c, cat_c, adj, mask_item,
                             av, w1a, w1b, g_w2, g_w3,
                             bc=bc, boff=c * bc))
    if NCHUNK == 1:
        return outs[0]
    return jnp.concatenate(outs, axis=0)


# speedup vs baseline: 3.6719x; 1.0606x over previous
import functools

import jax
import jax.numpy as jnp
from jax import lax
from jax.experimental import pallas as pl
from jax.experimental.pallas import tpu as pltpu
from jax.experimental.pallas import tpu_sc as plsc

B = 1024
L = 20
D = 128
S = 12
ALPHA = 0.2

NC = 2
NS = 16
NW = NC * NS
N_FLAT = B * L
CHUNK = N_FLAT // NW
TILE = 40
N_TILES = CHUNK // TILE
EB = 128
LANES = 16
NSEG = TILE * S // EB
NCHUNK = 4


def _leaky(x, slope):
    return jnp.maximum(x, slope * x)


GROUP = 8


def _make_sc_body(nflat_c, coff):
    per_worker = nflat_c // NW
    n_tiles = per_worker // TILE

    def body(inputs_hbm, item_hbm, cat_hbm, emb_hbm,
             h_out, ite_out, cat_out, ent_out,
             idx_v, iidx_v, h_v, ite_v, cat_v, ent_v,
             sem_h, sem_a, sem_e):
        wid = lax.axis_index("s") * NC + lax.axis_index("c")
        base = wid * per_worker

        def tile_body(ti, carry):
            gbase = base + ti * TILE
            pltpu.sync_copy(inputs_hbm.at[pl.ds(coff + gbase, TILE)], idx_v)
            pltpu.sync_copy(item_hbm.at[pl.ds(coff + gbase, TILE)], iidx_v)
            c_h = pltpu.async_copy(emb_hbm.at[idx_v], h_v, sem_h)
            c_it = pltpu.async_copy(emb_hbm.at[iidx_v], ite_v, sem_h)
            c_cat = pltpu.async_copy(cat_hbm.at[idx_v], cat_v, sem_a)
            c_cat.wait()

            def fire(g, fcarry):
                for i in range(GROUP):
                    r = g * GROUP + i
                    pltpu.async_copy(
                        emb_hbm.at[cat_v.at[r].at[pl.ds(0, S)]],
                        ent_v.at[r], sem_e)
                return fcarry

            lax.fori_loop(0, TILE // GROUP, fire, 0)
            c_h.wait()
            c_it.wait()
            pltpu.sync_copy(h_v, h_out.at[pl.ds(gbase, TILE)])
            pltpu.sync_copy(ite_v, ite_out.at[pl.ds(gbase, TILE)])
            pltpu.sync_copy(cat_v, cat_out.at[pl.ds(gbase, TILE)])

            def drain(g, dcarry):
                pltpu.make_async_copy(
                    emb_hbm.at[pl.ds(0, 8)],
                    ent_v.at[0].at[pl.ds(0, 8)], sem_e).wait()
                return dcarry

            lax.fori_loop(0, TILE * S * D * 4 // 4096, drain, 0)
            pltpu.sync_copy(ent_v, ent_out.at[pl.ds(gbase, TILE)])
            return carry

        lax.fori_loop(0, n_tiles, tile_body, 0)

    return body


@functools.cache
def _make_sc_gather(nflat_c, coff):
    return pl.kernel(
        _make_sc_body(nflat_c, coff),
        out_type=(
            jax.ShapeDtypeStruct((nflat_c, D), jnp.float32),
            jax.ShapeDtypeStruct((nflat_c, D), jnp.float32),
            jax.ShapeDtypeStruct((nflat_c, 128), jnp.int32),
            jax.ShapeDtypeStruct((nflat_c, S, D), jnp.float32),
        ),
        mesh=plsc.VectorSubcoreMesh(core_axis_name="c", subcore_axis_name="s",
                                    num_cores=NC, num_subcores=NS),
        scratch_types=[
            pltpu.VMEM((TILE,), jnp.int32),
            pltpu.VMEM((TILE,), jnp.int32),
            pltpu.VMEM((TILE, D), jnp.float32),
            pltpu.VMEM((TILE, D), jnp.float32),
            pltpu.VMEM((TILE, 128), jnp.int32),
            pltpu.VMEM((TILE, S, D), jnp.float32),
            pltpu.SemaphoreType.DMA,
            pltpu.SemaphoreType.DMA,
            pltpu.SemaphoreType.DMA,
        ],
    )


BB = 32


def _tc_body(h_ref, ite_ref, ent_ref, cat_ref, adj_ref, mask_ref,
             a_ref, w1a_ref, w1b_ref, w2_ref, w3_ref, o_ref):
    hflat = h_ref[...]
    hb = hflat.reshape(BB, L, D)
    av = a_ref[...]
    ha = hb[:, :, None, :] * av[None, None, :, :]
    e = lax.dot_general(ha.reshape(BB, L * 4, D), hb,
                        (((2,), (2,)), ((0,), (0,))))
    e = _leaky(e, ALPHA).reshape(BB, L, 4, L)
    adj = adj_ref[...]
    neg = jnp.float32(-9e15)
    att = jnp.where(adj == 1, e[:, :, 0, :], neg)
    att = jnp.where(adj == 2, e[:, :, 1, :], att)
    att = jnp.where(adj == 3, e[:, :, 2, :], att)
    att = jnp.where(adj == 4, e[:, :, 3, :], att)
    att = jax.nn.softmax(att, axis=-1)
    h_local = lax.dot_general(att, hb,
                              (((2,), (1,)), ((0,), (0,))))

    maskf = mask_ref[...].astype(jnp.float32)
    ite = ite_ref[...].reshape(BB, L, D)
    ssum = jnp.sum(ite * maskf[..., None], axis=1)
    sess = ssum / jnp.sum(maskf, axis=1)[..., None]

    catv = cat_ref[...]
    sess_pos = jnp.broadcast_to(
        sess[:, None, :], (BB, L, D)).reshape(BB * L, D)
    w1b2 = w1b_ref[...]
    ents = []
    logits = []
    for s in range(S):
        ent_s = ent_ref[:, s, :]
        wv_s = lax.bitcast_convert_type(catv[:, S + s:S + s + 1],
                                        jnp.float32)
        al_s = jnp.dot(sess_pos * ent_s, w1a_ref[...],
                       preferred_element_type=jnp.float32)
        al_s = _leaky(al_s + wv_s * w1b2, 0.2)
        logit_s = jnp.dot(al_s, w2_ref[...],
                          preferred_element_type=jnp.float32)
        ents.append(ent_s)
        logits.append(logit_s)
    m = logits[0]
    for s in range(1, S):
        m = jnp.maximum(m, logits[s])
    exps = [jnp.exp(lg - m) for lg in logits]
    denom = exps[0]
    for s in range(1, S):
        denom = denom + exps[s]
    nbr = (exps[0] / denom) * ents[0]
    for s in range(1, S):
        nbr = nbr + (exps[s] / denom) * ents[s]
    out = jnp.concatenate([hflat, nbr], axis=-1)
    hg = jnp.dot(out, w3_ref[...], preferred_element_type=jnp.float32)
    hg = jnp.maximum(hg, 0.0)
    o_ref[...] = h_local + hg.reshape(BB, L, D)


def _tc_call(h2, ite2, ent3, cat2, adj, mask, av, w1a, w1b, g_w2, g_w3,
             bc=B, boff=0, interpret=False):
    nblk = bc // BB
    ob = boff // BB
    return pl.pallas_call(
        _tc_body,
        grid=(nblk,),
        in_specs=[
            pl.BlockSpec((BB * L, D), lambda b: (b, 0)),
            pl.BlockSpec((BB * L, D), lambda b: (b, 0)),
            pl.BlockSpec((BB * L, S, D), lambda b: (b, 0, 0)),
            pl.BlockSpec((BB * L, 128), lambda b: (b, 0)),
            pl.BlockSpec((BB, L, L), lambda b, _o=ob: (b + _o, 0, 0)),
            pl.BlockSpec((BB, L), lambda b, _o=ob: (b + _o, 0)),
            pl.BlockSpec((4, D), lambda b: (0, 0)),
            pl.BlockSpec((D, D), lambda b: (0, 0)),
            pl.BlockSpec((1, D), lambda b: (0, 0)),
            pl.BlockSpec((D, 1), lambda b: (0, 0)),
            pl.BlockSpec((2 * D, D), lambda b: (0, 0)),
        ],
        out_specs=pl.BlockSpec((BB, L, D), lambda b: (b, 0, 0)),
        out_shape=jax.ShapeDtypeStruct((bc, L, D), jnp.float32),
        interpret=interpret,
    )(h2, ite2, ent3, cat2, adj, mask, av, w1a, w1b, g_w2, g_w3)


def kernel(inputs, adj, mask_item, item, adj_all, num_w, embedding,
           a_0, a_1, a_2, a_3, g_w1, g_w2, g_w3):
    flat = inputs.reshape(-1)
    itemf = item.reshape(-1)
    numw_bits = lax.bitcast_convert_type(num_w, jnp.int32)
    cat = jnp.concatenate(
        [adj_all, numw_bits,
         jnp.zeros((adj_all.shape[0], 128 - 2 * S), jnp.int32)], axis=1)
    av = jnp.concatenate([a_0, a_1, a_2, a_3], axis=1).T
    w1a = g_w1[:D]
    w1b = g_w1[D:]

    bc = B // NCHUNK
    nflat_c = bc * L
    outs = []
    for c in range(NCHUNK):
        h_c, ite_c, cat_c, ent_c = _make_sc_gather(nflat_c, c * nflat_c)(
            flat, itemf, cat, embedding)
        outs.append(_tc_call(h_c, ite_c, ent_c, cat_c, adj, mask_item,
                             av, w1a, w1b, g_w2, g_w3,
                             bc=bc, boff=c * bc))
    if NCHUNK == 1:
        return outs[0]
    return jnp.concatenate(outs, axis=0)


# speedup vs baseline: 3.6807x; 1.0024x over previous
import functools

import jax
import jax.numpy as jnp
from jax import lax
from jax.experimental import pallas as pl
from jax.experimental.pallas import tpu as pltpu
from jax.experimental.pallas import tpu_sc as plsc

B = 1024
L = 20
D = 128
S = 12
ALPHA = 0.2

NC = 2
NS = 16
NW = NC * NS
N_FLAT = B * L
CHUNK = N_FLAT // NW
TILE = 40
N_TILES = CHUNK // TILE
EB = 128
LANES = 16
NSEG = TILE * S // EB
NCHUNK = 4


def _leaky(x, slope):
    return jnp.maximum(x, slope * x)


GROUP = 8


def _make_sc_body(nflat_c, coff):
    per_worker = nflat_c // NW
    n_tiles = per_worker // TILE

    def body(inputs_hbm, item_hbm, cat_hbm, emb_hbm,
             h_out, ite_out, cat_out, ent_out,
             idx_v, iidx_v, h_v, ite_v, cat_v, ent_v,
             sem_h, sem_a, sem_e):
        wid = lax.axis_index("s") * NC + lax.axis_index("c")
        base = wid * per_worker

        def tile_body(ti, carry):
            gbase = base + ti * TILE
            pltpu.sync_copy(inputs_hbm.at[pl.ds(coff + gbase, TILE)], idx_v)
            pltpu.sync_copy(item_hbm.at[pl.ds(coff + gbase, TILE)], iidx_v)
            c_h = pltpu.async_copy(emb_hbm.at[idx_v], h_v, sem_h)
            c_it = pltpu.async_copy(emb_hbm.at[iidx_v], ite_v, sem_h)
            c_cat = pltpu.async_copy(cat_hbm.at[idx_v], cat_v, sem_a)
            c_cat.wait()

            def fire(g, fcarry):
                for i in range(GROUP):
                    r = g * GROUP + i
                    pltpu.async_copy(
                        emb_hbm.at[cat_v.at[r].at[pl.ds(0, S)]],
                        ent_v.at[r], sem_e)
                return fcarry

            lax.fori_loop(0, TILE // GROUP, fire, 0)
            c_h.wait()
            c_it.wait()
            pltpu.sync_copy(h_v, h_out.at[pl.ds(gbase, TILE)])
            pltpu.sync_copy(ite_v, ite_out.at[pl.ds(gbase, TILE)])
            pltpu.sync_copy(cat_v, cat_out.at[pl.ds(gbase, TILE)])

            def drain(g, dcarry):
                pltpu.make_async_copy(
                    emb_hbm.at[pl.ds(0, 8)],
                    ent_v.at[0].at[pl.ds(0, 8)], sem_e).wait()
                return dcarry

            lax.fori_loop(0, TILE * S * D * 4 // 4096, drain, 0)
            pltpu.sync_copy(ent_v, ent_out.at[pl.ds(gbase, TILE)])
            return carry

        lax.fori_loop(0, n_tiles, tile_body, 0)

    return body


@functools.cache
def _make_sc_gather(nflat_c, coff):
    return pl.kernel(
        _make_sc_body(nflat_c, coff),
        out_type=(
            jax.ShapeDtypeStruct((nflat_c, D), jnp.float32),
            jax.ShapeDtypeStruct((nflat_c, D), jnp.float32),
            jax.ShapeDtypeStruct((nflat_c, 128), jnp.int32),
            jax.ShapeDtypeStruct((nflat_c, S, D), jnp.float32),
        ),
        mesh=plsc.VectorSubcoreMesh(core_axis_name="c", subcore_axis_name="s",
                                    num_cores=NC, num_subcores=NS),
        scratch_types=[
            pltpu.VMEM((TILE,), jnp.int32),
            pltpu.VMEM((TILE,), jnp.int32),
            pltpu.VMEM((TILE, D), jnp.float32),
            pltpu.VMEM((TILE, D), jnp.float32),
            pltpu.VMEM((TILE, 128), jnp.int32),
            pltpu.VMEM((TILE, S, D), jnp.float32),
            pltpu.SemaphoreType.DMA,
            pltpu.SemaphoreType.DMA,
            pltpu.SemaphoreType.DMA,
        ],
    )


BB = 32


def _tc_body(h_ref, ite_ref, ent_ref, cat_ref, adj_ref, mask_ref,
             a_ref, w1a_ref, w1b_ref, w2_ref, w3_ref, obuf_ref, o_ref):
    del obuf_ref
    hflat = h_ref[...]
    hb = hflat.reshape(BB, L, D)
    av = a_ref[...]
    ha = hb[:, :, None, :] * av[None, None, :, :]
    e = lax.dot_general(ha.reshape(BB, L * 4, D), hb,
                        (((2,), (2,)), ((0,), (0,))))
    e = _leaky(e, ALPHA).reshape(BB, L, 4, L)
    adj = adj_ref[...]
    neg = jnp.float32(-9e15)
    att = jnp.where(adj == 1, e[:, :, 0, :], neg)
    att = jnp.where(adj == 2, e[:, :, 1, :], att)
    att = jnp.where(adj == 3, e[:, :, 2, :], att)
    att = jnp.where(adj == 4, e[:, :, 3, :], att)
    att = jax.nn.softmax(att, axis=-1)
    h_local = lax.dot_general(att, hb,
                              (((2,), (1,)), ((0,), (0,))))

    maskf = mask_ref[...].astype(jnp.float32)
    ite = ite_ref[...].reshape(BB, L, D)
    ssum = jnp.sum(ite * maskf[..., None], axis=1)
    sess = ssum / jnp.sum(maskf, axis=1)[..., None]

    catv = cat_ref[...]
    sess_pos = jnp.broadcast_to(
        sess[:, None, :], (BB, L, D)).reshape(BB * L, D)
    w1b2 = w1b_ref[...]
    ents = []
    logits = []
    for s in range(S):
        ent_s = ent_ref[:, s, :]
        wv_s = lax.bitcast_convert_type(catv[:, S + s:S + s + 1],
                                        jnp.float32)
        al_s = jnp.dot(sess_pos * ent_s, w1a_ref[...],
                       preferred_element_type=jnp.float32)
        al_s = _leaky(al_s + wv_s * w1b2, 0.2)
        logit_s = jnp.dot(al_s, w2_ref[...],
                          preferred_element_type=jnp.float32)
        ents.append(ent_s)
        logits.append(logit_s)
    m = logits[0]
    for s in range(1, S):
        m = jnp.maximum(m, logits[s])
    exps = [jnp.exp(lg - m) for lg in logits]
    denom = exps[0]
    for s in range(1, S):
        denom = denom + exps[s]
    nbr = (exps[0] / denom) * ents[0]
    for s in range(1, S):
        nbr = nbr + (exps[s] / denom) * ents[s]
    out = jnp.concatenate([hflat, nbr], axis=-1)
    hg = jnp.dot(out, w3_ref[...], preferred_element_type=jnp.float32)
    hg = jnp.maximum(hg, 0.0)
    o_ref[...] = h_local + hg.reshape(BB, L, D)


def _tc_call(h2, ite2, ent3, cat2, adj, mask, av, w1a, w1b, g_w2, g_w3,
             obuf, bc=B, boff=0, interpret=False):
    nblk = bc // BB
    ob = boff // BB
    return pl.pallas_call(
        _tc_body,
        grid=(nblk,),
        in_specs=[
            pl.BlockSpec((BB * L, D), lambda b: (b, 0)),
            pl.BlockSpec((BB * L, D), lambda b: (b, 0)),
            pl.BlockSpec((BB * L, S, D), lambda b: (b, 0, 0)),
            pl.BlockSpec((BB * L, 128), lambda b: (b, 0)),
            pl.BlockSpec((BB, L, L), lambda b, _o=ob: (b + _o, 0, 0)),
            pl.BlockSpec((BB, L), lambda b, _o=ob: (b + _o, 0)),
            pl.BlockSpec((4, D), lambda b: (0, 0)),
            pl.BlockSpec((D, D), lambda b: (0, 0)),
            pl.BlockSpec((1, D), lambda b: (0, 0)),
            pl.BlockSpec((D, 1), lambda b: (0, 0)),
            pl.BlockSpec((2 * D, D), lambda b: (0, 0)),
            pl.BlockSpec(memory_space=pl.ANY),
        ],
        out_specs=pl.BlockSpec((BB, L, D), lambda b, _o=ob: (b + _o, 0, 0)),
        out_shape=jax.ShapeDtypeStruct((B, L, D), jnp.float32),
        input_output_aliases={11: 0},
        interpret=interpret,
    )(h2, ite2, ent3, cat2, adj, mask, av, w1a, w1b, g_w2, g_w3, obuf)


def kernel(inputs, adj, mask_item, item, adj_all, num_w, embedding,
           a_0, a_1, a_2, a_3, g_w1, g_w2, g_w3):
    flat = inputs.reshape(-1)
    itemf = item.reshape(-1)
    numw_bits = lax.bitcast_convert_type(num_w, jnp.int32)
    cat = (jnp.pad(adj_all, ((0, 0), (0, 128 - S)))
           + jnp.pad(numw_bits, ((0, 0), (S, 128 - 2 * S))))
    av = jnp.concatenate([a_0, a_1, a_2, a_3], axis=1).T
    w1a = g_w1[:D]
    w1b = g_w1[D:]

    bc = B // NCHUNK
    nflat_c = bc * L
    out = jnp.zeros((B, L, D), jnp.float32)
    for c in range(NCHUNK):
        h_c, ite_c, cat_c, ent_c = _make_sc_gather(nflat_c, c * nflat_c)(
            flat, itemf, cat, embedding)
        out = _tc_call(h_c, ite_c, ent_c, cat_c, adj, mask_item,
                       av, w1a, w1b, g_w2, g_w3, out,
                       bc=bc, boff=c * bc)
    return out


# speedup vs baseline: 3.6827x; 1.0006x over previous
import functools

import jax
import jax.numpy as jnp
from jax import lax
from jax.experimental import pallas as pl
from jax.experimental.pallas import tpu as pltpu
from jax.experimental.pallas import tpu_sc as plsc

B = 1024
L = 20
D = 128
S = 12
ALPHA = 0.2

NC = 2
NS = 16
NW = NC * NS
N_FLAT = B * L
CHUNK = N_FLAT // NW
TILE = 40
N_TILES = CHUNK // TILE
EB = 128
LANES = 16
NSEG = TILE * S // EB
NCHUNK = 4


def _leaky(x, slope):
    return jnp.maximum(x, slope * x)


GROUP = 8


def _make_sc_body(nflat_c, coff):
    per_worker = nflat_c // NW
    n_tiles = per_worker // TILE

    def body(inputs_hbm, item_hbm, cat_hbm, emb_hbm,
             h_out, ite_out, cat_out, ent_out,
             idx_v, iidx_v, h_v, ite_v, cat_v, ent_v,
             sem_h, sem_a, sem_e):
        wid = lax.axis_index("s") * NC + lax.axis_index("c")
        base = wid * per_worker

        def tile_body(ti, carry):
            gbase = base + ti * TILE
            pltpu.sync_copy(inputs_hbm.at[pl.ds(coff + gbase, TILE)], idx_v)
            pltpu.sync_copy(item_hbm.at[pl.ds(coff + gbase, TILE)], iidx_v)
            c_h = pltpu.async_copy(emb_hbm.at[idx_v], h_v, sem_h)
            c_it = pltpu.async_copy(emb_hbm.at[iidx_v], ite_v, sem_h)
            c_cat = pltpu.async_copy(cat_hbm.at[idx_v], cat_v, sem_a)
            c_cat.wait()

            def fire(g, fcarry):
                for i in range(GROUP):
                    r = g * GROUP + i
                    pltpu.async_copy(
                        emb_hbm.at[cat_v.at[r].at[pl.ds(0, S)]],
                        ent_v.at[r], sem_e)
                return fcarry

            lax.fori_loop(0, TILE // GROUP, fire, 0)
            c_h.wait()
            c_it.wait()
            pltpu.sync_copy(h_v, h_out.at[pl.ds(gbase, TILE)])
            pltpu.sync_copy(ite_v, ite_out.at[pl.ds(gbase, TILE)])
            pltpu.sync_copy(cat_v, cat_out.at[pl.ds(gbase, TILE)])

            def drain(g, dcarry):
                pltpu.make_async_copy(
                    emb_hbm.at[pl.ds(0, 8)],
                    ent_v.at[0].at[pl.ds(0, 8)], sem_e).wait()
                return dcarry

            lax.fori_loop(0, TILE * S * D * 4 // 4096, drain, 0)
            pltpu.sync_copy(ent_v, ent_out.at[pl.ds(gbase, TILE)])
            return carry

        lax.fori_loop(0, n_tiles, tile_body, 0)

    return body


@functools.cache
def _make_sc_gather(nflat_c, coff):
    return pl.kernel(
        _make_sc_body(nflat_c, coff),
        out_type=(
            jax.ShapeDtypeStruct((nflat_c, D), jnp.float32),
            jax.ShapeDtypeStruct((nflat_c, D), jnp.float32),
            jax.ShapeDtypeStruct((nflat_c, 128), jnp.int32),
            jax.ShapeDtypeStruct((nflat_c, S, D), jnp.float32),
        ),
        mesh=plsc.VectorSubcoreMesh(core_axis_name="c", subcore_axis_name="s",
                                    num_cores=NC, num_subcores=NS),
        scratch_types=[
            pltpu.VMEM((TILE,), jnp.int32),
            pltpu.VMEM((TILE,), jnp.int32),
            pltpu.VMEM((TILE, D), jnp.float32),
            pltpu.VMEM((TILE, D), jnp.float32),
            pltpu.VMEM((TILE, 128), jnp.int32),
            pltpu.VMEM((TILE, S, D), jnp.float32),
            pltpu.SemaphoreType.DMA,
            pltpu.SemaphoreType.DMA,
            pltpu.SemaphoreType.DMA,
        ],
    )


BB = 32


def _tc_body(h_ref, ite_ref, ent_ref, cat_ref, adj_ref, mask_ref,
             a_ref, w1a_ref, w1b_ref, w2_ref, w3_ref, obuf_ref, o_ref):
    del obuf_ref
    hflat = h_ref[...]
    hb = hflat.reshape(BB, L, D)
    av = a_ref[...]
    ha = hb[:, :, None, :] * av[None, None, :, :]
    e = lax.dot_general(ha.reshape(BB, L * 4, D), hb,
                        (((2,), (2,)), ((0,), (0,))))
    e = _leaky(e, ALPHA).reshape(BB, L, 4, L)
    adj = adj_ref[...]
    neg = jnp.float32(-9e15)
    att = jnp.where(adj == 1, e[:, :, 0, :], neg)
    att = jnp.where(adj == 2, e[:, :, 1, :], att)
    att = jnp.where(adj == 3, e[:, :, 2, :], att)
    att = jnp.where(adj == 4, e[:, :, 3, :], att)
    att = jnp.exp(att - jnp.max(att, axis=-1, keepdims=True))
    att = att * (1.0 / jnp.sum(att, axis=-1, keepdims=True))
    h_local = lax.dot_general(att, hb,
                              (((2,), (1,)), ((0,), (0,))))

    maskf = mask_ref[...].astype(jnp.float32)
    ite = ite_ref[...].reshape(BB, L, D)
    ssum = jnp.sum(ite * maskf[..., None], axis=1)
    sess = ssum / jnp.sum(maskf, axis=1)[..., None]

    catv = cat_ref[...]
    sess_pos = jnp.broadcast_to(
        sess[:, None, :], (BB, L, D)).reshape(BB * L, D)
    w1b2 = w1b_ref[...]
    ents = []
    logits = []
    for s in range(S):
        ent_s = ent_ref[:, s, :]
        wv_s = lax.bitcast_convert_type(catv[:, S + s:S + s + 1],
                                        jnp.float32)
        al_s = jnp.dot(sess_pos * ent_s, w1a_ref[...],
                       preferred_element_type=jnp.float32)
        al_s = _leaky(al_s + wv_s * w1b2, 0.2)
        logit_s = jnp.dot(al_s, w2_ref[...],
                          preferred_element_type=jnp.float32)
        ents.append(ent_s)
        logits.append(logit_s)
    m = logits[0]
    for s in range(1, S):
        m = jnp.maximum(m, logits[s])
    exps = [jnp.exp(lg - m) for lg in logits]
    denom = exps[0]
    for s in range(1, S):
        denom = denom + exps[s]
    rec = 1.0 / denom
    nbr = (exps[0] * rec) * ents[0]
    for s in range(1, S):
        nbr = nbr + (exps[s] * rec) * ents[s]
    out = jnp.concatenate([hflat, nbr], axis=-1)
    hg = jnp.dot(out, w3_ref[...], preferred_element_type=jnp.float32)
    hg = jnp.maximum(hg, 0.0)
    o_ref[...] = h_local + hg.reshape(BB, L, D)


def _tc_call(h2, ite2, ent3, cat2, adj, mask, av, w1a, w1b, g_w2, g_w3,
             obuf, bc=B, boff=0, interpret=False):
    nblk = bc // BB
    ob = boff // BB
    return pl.pallas_call(
        _tc_body,
        grid=(nblk,),
        in_specs=[
            pl.BlockSpec((BB * L, D), lambda b: (b, 0)),
            pl.BlockSpec((BB * L, D), lambda b: (b, 0)),
            pl.BlockSpec((BB * L, S, D), lambda b: (b, 0, 0)),
            pl.BlockSpec((BB * L, 128), lambda b: (b, 0)),
            pl.BlockSpec((BB, L, L), lambda b, _o=ob: (b + _o, 0, 0)),
            pl.BlockSpec((BB, L), lambda b, _o=ob: (b + _o, 0)),
            pl.BlockSpec((4, D), lambda b: (0, 0)),
            pl.BlockSpec((D, D), lambda b: (0, 0)),
            pl.BlockSpec((1, D), lambda b: (0, 0)),
            pl.BlockSpec((D, 1), lambda b: (0, 0)),
            pl.BlockSpec((2 * D, D), lambda b: (0, 0)),
            pl.BlockSpec(memory_space=pl.ANY),
        ],
        out_specs=pl.BlockSpec((BB, L, D), lambda b, _o=ob: (b + _o, 0, 0)),
        out_shape=jax.ShapeDtypeStruct((B, L, D), jnp.float32),
        input_output_aliases={11: 0},
        interpret=interpret,
    )(h2, ite2, ent3, cat2, adj, mask, av, w1a, w1b, g_w2, g_w3, obuf)


def kernel(inputs, adj, mask_item, item, adj_all, num_w, embedding,
           a_0, a_1, a_2, a_3, g_w1, g_w2, g_w3):
    flat = inputs.reshape(-1)
    itemf = item.reshape(-1)
    numw_bits = lax.bitcast_convert_type(num_w, jnp.int32)
    cat = (jnp.pad(adj_all, ((0, 0), (0, 128 - S)))
           + jnp.pad(numw_bits, ((0, 0), (S, 128 - 2 * S))))
    av = jnp.concatenate([a_0, a_1, a_2, a_3], axis=1).T
    w1a = g_w1[:D]
    w1b = g_w1[D:]

    bc = B // NCHUNK
    nflat_c = bc * L
    out = jnp.zeros((B, L, D), jnp.float32)
    for c in range(NCHUNK):
        h_c, ite_c, cat_c, ent_c = _make_sc_gather(nflat_c, c * nflat_c)(
            flat, itemf, cat, embedding)
        out = _tc_call(h_c, ite_c, ent_c, cat_c, adj, mask_item,
                       av, w1a, w1b, g_w2, g_w3, out,
                       bc=bc, boff=c * bc)
    return out
